# Initial kernel scaffold; baseline (speedup 1.0000x reference)
import jax
import jax.numpy as jnp
from jax.experimental import pallas as pl


def kernel(x, h, edge_index, W_rx, b_rx, W_rh, b_rh, W_ux, b_ux, W_uh, b_uh, W_cx, b_cx, W_ch, b_ch):
    raise NotImplementedError("write your pallas kernel here")


# speedup vs baseline: 5.9828x; 5.9828x over previous
import functools

import jax
import jax.numpy as jnp
from jax import lax
from jax.experimental import pallas as pl
from jax.experimental.pallas import tpu as pltpu
from jax.experimental.pallas import tpu_sc as plsc

N = 10000
E = 320000
D = 128

CW = 125
NCHUNK = E // CW
NC, NS = 2, 16
NP = 10240
ROWS_PER_TILE_N = NP // NS


def _mesh():
    return plsc.VectorSubcoreMesh(core_axis_name="c", subcore_axis_name="s")


SB = 16


def _sc_pass_xh(xh, src2, dst, zeros):
    rows_per_tile = NCHUNK // NS

    @functools.partial(
        pl.kernel,
        out_type=jax.ShapeDtypeStruct((2, NP, D), jnp.float32),
        mesh=_mesh(),
        scratch_types=[
            pltpu.VMEM_SHARED((NP, D), jnp.float32),
            pltpu.VMEM((SB, CW), jnp.int32),
            pltpu.VMEM((SB, CW), jnp.int32),
            pltpu.VMEM((CW, D), jnp.float32),
        ],
    )
    def k(xh_hbm, src_hbm, dst_hbm, zeros_hbm, out_hbm, acc, srcv, dstv, rowsv):
        cid = lax.axis_index("c")
        sid = lax.axis_index("s")
        nslice = pl.ds(sid * ROWS_PER_TILE_N, ROWS_PER_TILE_N)
        pltpu.sync_copy(zeros_hbm.at[nslice], acc.at[nslice])
        ebase = sid * rows_per_tile
        plsc.subcore_barrier()

        def outer(o, carry):
            pltpu.sync_copy(src_hbm.at[cid, pl.ds(ebase + o * SB, SB)], srcv)
            pltpu.sync_copy(dst_hbm.at[pl.ds(ebase + o * SB, SB)], dstv)

            def body(j, c2):
                pltpu.sync_copy(xh_hbm.at[srcv.at[j]], rowsv)
                pltpu.sync_copy(rowsv, acc.at[dstv.at[j]], add=True)
                return c2

            return lax.fori_loop(0, SB, body, carry)

        lax.fori_loop(0, rows_per_tile // SB, outer, 0)
        plsc.subcore_barrier()
        pltpu.sync_copy(acc.at[nslice], out_hbm.at[cid, nslice])

    return k(xh, src2, dst, zeros)


def _sc_pass_hh(hh, src, dst, zeros):
    rows_per_tile = NCHUNK // (NC * NS)

    @functools.partial(
        pl.kernel,
        out_type=jax.ShapeDtypeStruct((2, NP, D), jnp.float32),
        mesh=_mesh(),
        scratch_types=[
            pltpu.VMEM_SHARED((NP, D), jnp.float32),
            pltpu.VMEM((SB, CW), jnp.int32),
            pltpu.VMEM((SB, CW), jnp.int32),
            pltpu.VMEM((CW, D), jnp.float32),
        ],
    )
    def k(hh_hbm, src_hbm, dst_hbm, zeros_hbm, out_hbm, acc, srcv, dstv, rowsv):
        cid = lax.axis_index("c")
        sid = lax.axis_index("s")
        nslice = pl.ds(sid * ROWS_PER_TILE_N, ROWS_PER_TILE_N)
        pltpu.sync_copy(zeros_hbm.at[nslice], acc.at[nslice])
        ebase = (cid * NS + sid) * rows_per_tile
        plsc.subcore_barrier()

        def outer(o, carry):
            pltpu.sync_copy(src_hbm.at[pl.ds(ebase + o * SB, SB)], srcv)
            pltpu.sync_copy(dst_hbm.at[pl.ds(ebase + o * SB, SB)], dstv)

            def body(j, c2):
                pltpu.sync_copy(hh_hbm.at[srcv.at[j]], rowsv)
                pltpu.sync_copy(rowsv, acc.at[dstv.at[j]], add=True)
                return c2

            return lax.fori_loop(0, SB, body, carry)

        lax.fori_loop(0, rows_per_tile // SB, outer, 0)
        plsc.subcore_barrier()
        pltpu.sync_copy(acc.at[nslice], out_hbm.at[cid, nslice])

    return k(hh, src, dst, zeros)


_BLK = 1000


def _tc_gates(aggxh, h, w_rx, w_rh, w_ux, w_uh, w_cx, b_r, b_u, b_cx):
    def body(ax, ah, h_ref, wrx, wrh, wux, wuh, wcx, br, bu, bcx, u_out, hh_out, cxp_out):
        axv = ax[...]
        ahv = ah[...]
        dot = functools.partial(
            jnp.dot, preferred_element_type=jnp.float32, precision=lax.Precision.HIGHEST
        )
        r = jax.nn.sigmoid(dot(axv, wrx[...]) + dot(ahv, wrh[...]) + br[...])
        u = jax.nn.sigmoid(dot(axv, wux[...]) + dot(ahv, wuh[...]) + bu[...])
        u_out[...] = u
        hh_out[...] = r * h_ref[...]
        cxp_out[...] = dot(axv, wcx[...]) + bcx[...]

    grid = (N // _BLK,)
    row_spec = pl.BlockSpec((_BLK, D), lambda i: (i, 0))
    w_spec = pl.BlockSpec((D, D), lambda i: (0, 0))
    b_spec = pl.BlockSpec((1, D), lambda i: (0, 0))
    ax_spec = pl.BlockSpec((1, _BLK, D), lambda i: (0, i, 0))
    ah_spec = pl.BlockSpec((1, _BLK, D), lambda i: (1, i, 0))

    def body2(ax, ah, h_ref, wrx, wrh, wux, wuh, wcx, br, bu, bcx, u_out, hh_out, cxp_out):
        body(ax.at[0], ah.at[0], h_ref, wrx, wrh, wux, wuh, wcx, br, bu, bcx,
             u_out, hh_out, cxp_out)

    return pl.pallas_call(
        body2,
        grid=grid,
        in_specs=[ax_spec, ah_spec, row_spec, w_spec, w_spec, w_spec, w_spec,
                  w_spec, b_spec, b_spec, b_spec],
        out_specs=[row_spec, row_spec, row_spec],
        out_shape=[jax.ShapeDtypeStruct((N, D), jnp.float32)] * 3,
    )(aggxh, aggxh, h, w_rx, w_rh, w_ux, w_uh, w_cx, b_r, b_u, b_cx)


def _tc_final(cxp, parts, u, h, w_ch, b_ch):
    def body(cxp_ref, p_ref, u_ref, h_ref, wch, bch, out):
        agg = p_ref[0] + p_ref[1]
        dot = functools.partial(
            jnp.dot, preferred_element_type=jnp.float32, precision=lax.Precision.HIGHEST
        )
        c = jnp.tanh(cxp_ref[...] + dot(agg, wch[...]) + bch[...])
        uv = u_ref[...]
        out[...] = uv * h_ref[...] + (1.0 - uv) * c

    grid = (N // _BLK,)
    row_spec = pl.BlockSpec((_BLK, D), lambda i: (i, 0))
    p_spec = pl.BlockSpec((2, _BLK, D), lambda i: (0, i, 0))
    w_spec = pl.BlockSpec((D, D), lambda i: (0, 0))
    b_spec = pl.BlockSpec((1, D), lambda i: (0, 0))
    return pl.pallas_call(
        body,
        grid=grid,
        in_specs=[row_spec, p_spec, row_spec, row_spec, w_spec, b_spec],
        out_specs=row_spec,
        out_shape=jax.ShapeDtypeStruct((N, D), jnp.float32),
    )(cxp, parts, u, h, w_ch, b_ch)


def kernel(x, h, edge_index, W_rx, b_rx, W_rh, b_rh, W_ux, b_ux, W_uh, b_uh,
           W_cx, b_cx, W_ch, b_ch):
    src = edge_index[0]
    dst = edge_index[1]
    src2 = jnp.stack([src, src + N]).reshape(2, NCHUNK, CW)
    srcr = src.reshape(NCHUNK, CW)
    dstr = dst.reshape(NCHUNK, CW)
    xh = jnp.concatenate([x, h], axis=0)
    zeros = jnp.zeros((NP, D), jnp.float32)

    aggxh = _sc_pass_xh(xh, src2, dstr, zeros)

    b_r = (b_rx + b_rh).reshape(1, D)
    b_u = (b_ux + b_uh).reshape(1, D)
    b_c = (b_cx + b_ch).reshape(1, D)
    u, hh, cxp = _tc_gates(aggxh, h, W_rx, W_rh, W_ux, W_uh, W_cx,
                           b_r, b_u, b_c)

    parts = _sc_pass_hh(hh, srcr, dstr, zeros)

    zero_b = jnp.zeros((1, D), jnp.float32)
    return _tc_final(cxp, parts, u, h, W_ch, zero_b)


# speedup vs baseline: 9.0538x; 1.5133x over previous
import functools

import jax
import jax.numpy as jnp
from jax import lax
from jax.experimental import pallas as pl
from jax.experimental.pallas import tpu as pltpu
from jax.experimental.pallas import tpu_sc as plsc

N = 10000
E = 320000
D = 128

CW = 125
NCHUNK = E // CW
NC, NS = 2, 16
NP = 10240
ROWS_PER_TILE_N = NP // NS


def _mesh():
    return plsc.VectorSubcoreMesh(core_axis_name="c", subcore_axis_name="s")


SB = 16

_SC_SCRATCH = [
    pltpu.VMEM_SHARED((NP, D), jnp.float32),
    pltpu.VMEM((2, SB, CW), jnp.int32),
    pltpu.VMEM((2, SB, CW), jnp.int32),
    pltpu.VMEM((2, CW, D), jnp.float32),
    pltpu.SemaphoreType.DMA((2,)),
    pltpu.SemaphoreType.DMA((2,)),
]


def _edge_pipeline(table_hbm, src_block, dst_block, acc, srcv, dstv, rowsv,
                   sg, si, total):
    nblocks = total // SB

    def gather_desc(i):
        ob, j, b = i // SB, i % SB, i % 2
        return pltpu.make_async_copy(
            table_hbm.at[srcv.at[ob % 2, j]], rowsv.at[b], sg.at[b])

    pltpu.sync_copy(src_block(0), srcv.at[0])
    pltpu.sync_copy(dst_block(0), dstv.at[0])
    if nblocks > 1:
        pltpu.async_copy(src_block(1), srcv.at[1], si.at[1])
        pltpu.async_copy(dst_block(1), dstv.at[1], si.at[1])
    gather_desc(0).start()

    def body(i, carry):
        ob, j = i // SB, i % SB

        @pl.when(jnp.logical_and(i < total, j == 0))
        def _():
            p = ob % 2
            pltpu.make_async_copy(src_block(ob), srcv.at[p], si.at[p]).wait()
            pltpu.make_async_copy(dst_block(ob), dstv.at[p], si.at[p]).wait()

        @pl.when(i < total)
        def _():
            gather_desc(i).start()

        ip = i - 1
        obp, jp, bp = ip // SB, ip % SB, ip % 2
        gather_desc(ip).wait()
        pltpu.sync_copy(rowsv.at[bp], acc.at[dstv.at[obp % 2, jp]], add=True)

        @pl.when(jnp.logical_and(
            jnp.logical_and(i < total, j == 0), ob + 1 < nblocks))
        def _():
            pn = (ob + 1) % 2
            pltpu.async_copy(src_block(ob + 1), srcv.at[pn], si.at[pn])
            pltpu.async_copy(dst_block(ob + 1), dstv.at[pn], si.at[pn])

        return carry

    lax.fori_loop(1, total + 1, body, 0)


def _sc_pass_xh(xh, src2, dst, zeros):
    rows_per_tile = NCHUNK // NS

    @functools.partial(
        pl.kernel,
        out_type=jax.ShapeDtypeStruct((2, NP, D), jnp.float32),
        mesh=_mesh(),
        scratch_types=_SC_SCRATCH,
    )
    def k(xh_hbm, src_hbm, dst_hbm, zeros_hbm, out_hbm,
          acc, srcv, dstv, rowsv, sg, si):
        cid = lax.axis_index("c")
        sid = lax.axis_index("s")
        nslice = pl.ds(sid * ROWS_PER_TILE_N, ROWS_PER_TILE_N)
        pltpu.sync_copy(zeros_hbm.at[nslice], acc.at[nslice])
        ebase = sid * rows_per_tile
        plsc.subcore_barrier()

        _edge_pipeline(
            xh_hbm,
            lambda o: src_hbm.at[cid, pl.ds(ebase + o * SB, SB)],
            lambda o: dst_hbm.at[pl.ds(ebase + o * SB, SB)],
            acc, srcv, dstv, rowsv, sg, si, rows_per_tile)

        plsc.subcore_barrier()
        pltpu.sync_copy(acc.at[nslice], out_hbm.at[cid, nslice])

    return k(xh, src2, dst, zeros)


def _sc_pass_hh(hh, src, dst, zeros):
    rows_per_tile = NCHUNK // (NC * NS)

    @functools.partial(
        pl.kernel,
        out_type=jax.ShapeDtypeStruct((2, NP, D), jnp.float32),
        mesh=_mesh(),
        scratch_types=_SC_SCRATCH,
    )
    def k(hh_hbm, src_hbm, dst_hbm, zeros_hbm, out_hbm,
          acc, srcv, dstv, rowsv, sg, si):
        cid = lax.axis_index("c")
        sid = lax.axis_index("s")
        nslice = pl.ds(sid * ROWS_PER_TILE_N, ROWS_PER_TILE_N)
        pltpu.sync_copy(zeros_hbm.at[nslice], acc.at[nslice])
        ebase = (cid * NS + sid) * rows_per_tile
        plsc.subcore_barrier()

        _edge_pipeline(
            hh_hbm,
            lambda o: src_hbm.at[pl.ds(ebase + o * SB, SB)],
            lambda o: dst_hbm.at[pl.ds(ebase + o * SB, SB)],
            acc, srcv, dstv, rowsv, sg, si, rows_per_tile)

        plsc.subcore_barrier()
        pltpu.sync_copy(acc.at[nslice], out_hbm.at[cid, nslice])

    return k(hh, src, dst, zeros)


_BLK = 1000


def _tc_gates(aggxh, h, w_rx, w_rh, w_ux, w_uh, w_cx, b_r, b_u, b_cx):
    def body(ax, ah, h_ref, wrx, wrh, wux, wuh, wcx, br, bu, bcx, u_out, hh_out, cxp_out):
        axv = ax[...]
        ahv = ah[...]
        dot = functools.partial(
            jnp.dot, preferred_element_type=jnp.float32, precision=lax.Precision.HIGHEST
        )
        r = jax.nn.sigmoid(dot(axv, wrx[...]) + dot(ahv, wrh[...]) + br[...])
        u = jax.nn.sigmoid(dot(axv, wux[...]) + dot(ahv, wuh[...]) + bu[...])
        u_out[...] = u
        hh_out[...] = r * h_ref[...]
        cxp_out[...] = dot(axv, wcx[...]) + bcx[...]

    grid = (N // _BLK,)
    row_spec = pl.BlockSpec((_BLK, D), lambda i: (i, 0))
    w_spec = pl.BlockSpec((D, D), lambda i: (0, 0))
    b_spec = pl.BlockSpec((1, D), lambda i: (0, 0))
    ax_spec = pl.BlockSpec((1, _BLK, D), lambda i: (0, i, 0))
    ah_spec = pl.BlockSpec((1, _BLK, D), lambda i: (1, i, 0))

    def body2(ax, ah, h_ref, wrx, wrh, wux, wuh, wcx, br, bu, bcx, u_out, hh_out, cxp_out):
        body(ax.at[0], ah.at[0], h_ref, wrx, wrh, wux, wuh, wcx, br, bu, bcx,
             u_out, hh_out, cxp_out)

    return pl.pallas_call(
        body2,
        grid=grid,
        in_specs=[ax_spec, ah_spec, row_spec, w_spec, w_spec, w_spec, w_spec,
                  w_spec, b_spec, b_spec, b_spec],
        out_specs=[row_spec, row_spec, row_spec],
        out_shape=[jax.ShapeDtypeStruct((N, D), jnp.float32)] * 3,
    )(aggxh, aggxh, h, w_rx, w_rh, w_ux, w_uh, w_cx, b_r, b_u, b_cx)


def _tc_final(cxp, parts, u, h, w_ch, b_ch):
    def body(cxp_ref, p_ref, u_ref, h_ref, wch, bch, out):
        agg = p_ref[0] + p_ref[1]
        dot = functools.partial(
            jnp.dot, preferred_element_type=jnp.float32, precision=lax.Precision.HIGHEST
        )
        c = jnp.tanh(cxp_ref[...] + dot(agg, wch[...]) + bch[...])
        uv = u_ref[...]
        out[...] = uv * h_ref[...] + (1.0 - uv) * c

    grid = (N // _BLK,)
    row_spec = pl.BlockSpec((_BLK, D), lambda i: (i, 0))
    p_spec = pl.BlockSpec((2, _BLK, D), lambda i: (0, i, 0))
    w_spec = pl.BlockSpec((D, D), lambda i: (0, 0))
    b_spec = pl.BlockSpec((1, D), lambda i: (0, 0))
    return pl.pallas_call(
        body,
        grid=grid,
        in_specs=[row_spec, p_spec, row_spec, row_spec, w_spec, b_spec],
        out_specs=row_spec,
        out_shape=jax.ShapeDtypeStruct((N, D), jnp.float32),
    )(cxp, parts, u, h, w_ch, b_ch)


def kernel(x, h, edge_index, W_rx, b_rx, W_rh, b_rh, W_ux, b_ux, W_uh, b_uh,
           W_cx, b_cx, W_ch, b_ch):
    src = edge_index[0]
    dst = edge_index[1]
    src2 = jnp.stack([src, src + N]).reshape(2, NCHUNK, CW)
    srcr = src.reshape(NCHUNK, CW)
    dstr = dst.reshape(NCHUNK, CW)
    xh = jnp.concatenate([x, h], axis=0)
    zeros = jnp.zeros((NP, D), jnp.float32)

    aggxh = _sc_pass_xh(xh, src2, dstr, zeros)

    b_r = (b_rx + b_rh).reshape(1, D)
    b_u = (b_ux + b_uh).reshape(1, D)
    b_c = (b_cx + b_ch).reshape(1, D)
    u, hh, cxp = _tc_gates(aggxh, h, W_rx, W_rh, W_ux, W_uh, W_cx,
                           b_r, b_u, b_c)

    parts = _sc_pass_hh(hh, srcr, dstr, zeros)

    zero_b = jnp.zeros((1, D), jnp.float32)
    return _tc_final(cxp, parts, u, h, W_ch, zero_b)


# speedup vs baseline: 10.1727x; 1.1236x over previous
import functools

import jax
import jax.numpy as jnp
from jax import lax
from jax.experimental import pallas as pl
from jax.experimental.pallas import tpu as pltpu
from jax.experimental.pallas import tpu_sc as plsc

N = 10000
E = 320000
D = 128

CW = 125
NCHUNK = E // CW
NC, NS = 2, 16
NP = 10240
ROWS_PER_TILE_N = NP // NS


def _mesh():
    return plsc.VectorSubcoreMesh(core_axis_name="c", subcore_axis_name="s")


SB = 16

_SC_SCRATCH = [
    pltpu.VMEM_SHARED((NP, D), jnp.float32),
    pltpu.VMEM((2, SB, CW), jnp.int32),
    pltpu.VMEM((2, SB, CW), jnp.int32),
    pltpu.VMEM((2, CW, D), jnp.float32),
    pltpu.SemaphoreType.DMA((2,)),
    pltpu.SemaphoreType.DMA((2,)),
]


def _edge_pipeline(table_hbm, src_block, dst_block, acc, srcv, dstv, rowsv,
                   sg, si, total):
    nblocks = total // SB

    def gather_desc(i):
        ob, j, b = i // SB, i % SB, i % 2
        return pltpu.make_async_copy(
            table_hbm.at[srcv.at[ob % 2, j]], rowsv.at[b], sg.at[b])

    pltpu.sync_copy(src_block(0), srcv.at[0])
    pltpu.sync_copy(dst_block(0), dstv.at[0])
    if nblocks > 1:
        pltpu.async_copy(src_block(1), srcv.at[1], si.at[1])
        pltpu.async_copy(dst_block(1), dstv.at[1], si.at[1])
    gather_desc(0).start()

    def body(i, carry):
        ob, j = i // SB, i % SB

        @pl.when(jnp.logical_and(i < total, j == 0))
        def _():
            p = ob % 2
            pltpu.make_async_copy(src_block(ob), srcv.at[p], si.at[p]).wait()
            pltpu.make_async_copy(dst_block(ob), dstv.at[p], si.at[p]).wait()

        @pl.when(i < total)
        def _():
            gather_desc(i).start()

        ip = i - 1
        obp, jp, bp = ip // SB, ip % SB, ip % 2
        gather_desc(ip).wait()
        pltpu.sync_copy(rowsv.at[bp], acc.at[dstv.at[obp % 2, jp]], add=True)

        @pl.when(jnp.logical_and(
            jnp.logical_and(i < total, j == 0), ob + 1 < nblocks))
        def _():
            pn = (ob + 1) % 2
            pltpu.async_copy(src_block(ob + 1), srcv.at[pn], si.at[pn])
            pltpu.async_copy(dst_block(ob + 1), dstv.at[pn], si.at[pn])

        return carry

    lax.fori_loop(1, total + 1, body, 0)


def _sc_pass_xh(x, h, src, dst, zeros):
    rows_per_tile = NCHUNK // NS

    @functools.partial(
        pl.kernel,
        out_type=jax.ShapeDtypeStruct((2, NP, D), jnp.float32),
        mesh=_mesh(),
        scratch_types=_SC_SCRATCH,
    )
    def k(x_hbm, h_hbm, src_hbm, dst_hbm, zeros_hbm, out_hbm,
          acc, srcv, dstv, rowsv, sg, si):
        cid = lax.axis_index("c")
        sid = lax.axis_index("s")
        nslice = pl.ds(sid * ROWS_PER_TILE_N, ROWS_PER_TILE_N)
        pltpu.sync_copy(zeros_hbm.at[nslice], acc.at[nslice])
        ebase = sid * rows_per_tile
        src_block = lambda o: src_hbm.at[pl.ds(ebase + o * SB, SB)]
        dst_block = lambda o: dst_hbm.at[pl.ds(ebase + o * SB, SB)]
        plsc.subcore_barrier()

        @pl.when(cid == 0)
        def _():
            _edge_pipeline(x_hbm, src_block, dst_block,
                           acc, srcv, dstv, rowsv, sg, si, rows_per_tile)

        @pl.when(cid == 1)
        def _():
            _edge_pipeline(h_hbm, src_block, dst_block,
                           acc, srcv, dstv, rowsv, sg, si, rows_per_tile)

        plsc.subcore_barrier()
        pltpu.sync_copy(acc.at[nslice], out_hbm.at[cid, nslice])

    return k(x, h, src, dst, zeros)


def _sc_pass_hh(hh, src, dst, zeros):
    rows_per_tile = NCHUNK // (NC * NS)

    @functools.partial(
        pl.kernel,
        out_type=jax.ShapeDtypeStruct((2, NP, D), jnp.float32),
        mesh=_mesh(),
        scratch_types=_SC_SCRATCH,
    )
    def k(hh_hbm, src_hbm, dst_hbm, zeros_hbm, out_hbm,
          acc, srcv, dstv, rowsv, sg, si):
        cid = lax.axis_index("c")
        sid = lax.axis_index("s")
        nslice = pl.ds(sid * ROWS_PER_TILE_N, ROWS_PER_TILE_N)
        pltpu.sync_copy(zeros_hbm.at[nslice], acc.at[nslice])
        ebase = (cid * NS + sid) * rows_per_tile
        plsc.subcore_barrier()

        _edge_pipeline(
            hh_hbm,
            lambda o: src_hbm.at[pl.ds(ebase + o * SB, SB)],
            lambda o: dst_hbm.at[pl.ds(ebase + o * SB, SB)],
            acc, srcv, dstv, rowsv, sg, si, rows_per_tile)

        plsc.subcore_barrier()
        pltpu.sync_copy(acc.at[nslice], out_hbm.at[cid, nslice])

    return k(hh, src, dst, zeros)


_BLK = 1000


_dot = functools.partial(
    jnp.dot, preferred_element_type=jnp.float32, precision=lax.Precision.HIGHEST
)
_row_spec = pl.BlockSpec((_BLK, D), lambda i: (i, 0))
_w_spec = pl.BlockSpec((D, D), lambda i: (0, 0))
_b_spec = pl.BlockSpec((1, D), lambda i: (0, 0))
_ax_spec = pl.BlockSpec((1, _BLK, D), lambda i: (0, i, 0))
_ah_spec = pl.BlockSpec((1, _BLK, D), lambda i: (1, i, 0))


def _tc_r(aggxh, h, w_rx, w_rh, b_r):
    def body(ax, ah, h_ref, wrx, wrh, br, hh_out):
        r = jax.nn.sigmoid(
            _dot(ax[0], wrx[...]) + _dot(ah[0], wrh[...]) + br[...])
        hh_out[...] = r * h_ref[...]

    return pl.pallas_call(
        body,
        grid=(N // _BLK,),
        in_specs=[_ax_spec, _ah_spec, _row_spec, _w_spec, _w_spec, _b_spec],
        out_specs=_row_spec,
        out_shape=jax.ShapeDtypeStruct((N, D), jnp.float32),
    )(aggxh, aggxh, h, w_rx, w_rh, b_r)


def _tc_ucx(aggxh, w_ux, w_uh, w_cx, b_u, b_cx):
    def body(ax, ah, wux, wuh, wcx, bu, bcx, u_out, cxp_out):
        axv = ax[0]
        u_out[...] = jax.nn.sigmoid(
            _dot(axv, wux[...]) + _dot(ah[0], wuh[...]) + bu[...])
        cxp_out[...] = _dot(axv, wcx[...]) + bcx[...]

    return pl.pallas_call(
        body,
        grid=(N // _BLK,),
        in_specs=[_ax_spec, _ah_spec, _w_spec, _w_spec, _w_spec, _b_spec,
                  _b_spec],
        out_specs=[_row_spec, _row_spec],
        out_shape=[jax.ShapeDtypeStruct((N, D), jnp.float32)] * 2,
    )(aggxh, aggxh, w_ux, w_uh, w_cx, b_u, b_cx)


def _tc_final(cxp, parts, u, h, w_ch):
    def body(cxp_ref, p_ref, u_ref, h_ref, wch, out):
        agg = p_ref[0] + p_ref[1]
        c = jnp.tanh(cxp_ref[...] + _dot(agg, wch[...]))
        uv = u_ref[...]
        out[...] = uv * h_ref[...] + (1.0 - uv) * c

    p_spec = pl.BlockSpec((2, _BLK, D), lambda i: (0, i, 0))
    return pl.pallas_call(
        body,
        grid=(N // _BLK,),
        in_specs=[_row_spec, p_spec, _row_spec, _row_spec, _w_spec],
        out_specs=_row_spec,
        out_shape=jax.ShapeDtypeStruct((N, D), jnp.float32),
    )(cxp, parts, u, h, w_ch)


def kernel(x, h, edge_index, W_rx, b_rx, W_rh, b_rh, W_ux, b_ux, W_uh, b_uh,
           W_cx, b_cx, W_ch, b_ch):
    src = edge_index[0]
    dst = edge_index[1]
    srcr = src.reshape(NCHUNK, CW)
    dstr = dst.reshape(NCHUNK, CW)
    zeros = jnp.zeros((NP, D), jnp.float32)

    aggxh = _sc_pass_xh(x, h, srcr, dstr, zeros)

    b_r = (b_rx + b_rh).reshape(1, D)
    b_u = (b_ux + b_uh).reshape(1, D)
    b_c = (b_cx + b_ch).reshape(1, D)
    hh = _tc_r(aggxh, h, W_rx, W_rh, b_r)
    u, cxp = _tc_ucx(aggxh, W_ux, W_uh, W_cx, b_u, b_c)

    parts = _sc_pass_hh(hh, srcr, dstr, zeros)

    return _tc_final(cxp, parts, u, h, W_ch)
